# baseline (speedup 1.0000x reference)
import functools

import jax
import jax.numpy as jnp
from jax import lax
from jax.experimental import pallas as pl
from jax.experimental.pallas import tpu as pltpu
from jax.experimental.pallas import tpu_sc as plsc

N, E, D = 10000, 320000, 128

NC, NS = 2, 16
NW = NC * NS
CH = 128
NCHUNK = 80
E_PAD = NW * NCHUNK * CH
N_PAD = 10240
ROWS_PER_TILE = N_PAD // NS


def _agg_body(feats, idx3, out, idx16, ia, ib, rows_a, rows_b, acc_sh, sem):
    c = lax.axis_index("c")
    s = lax.axis_index("s")
    wid = c * NS + s
    r0 = pl.multiple_of(s * ROWS_PER_TILE, 8)

    zv = jnp.zeros((16,), jnp.float32)
    for k in range(16):
        for m in range(D // 16):
            rows_a[k, pl.ds(m * 16, 16)] = zv

    def zstep(k, _):
        pltpu.sync_copy(rows_a.at[pl.ds(0, 16)],
                        acc_sh.at[pl.ds(r0 + k * 16, 16)])
        return ()

    lax.fori_loop(0, ROWS_PER_TILE // 16, zstep, (), unroll=False)

    pltpu.sync_copy(idx3.at[wid], idx16)
    plsc.subcore_barrier()

    def unpack_chunk(i, dstbuf):
        for m in range(CH // 16):
            v = idx16[i, pl.ds(m * 16, 16)]
            dstbuf[0, pl.ds(m * 16, 16)] = lax.bitwise_and(v, jnp.int32(0xFFFF))
            dstbuf[1, pl.ds(m * 16, 16)] = lax.shift_right_logical(v, jnp.int32(16))

    def step(j, _):
        i0 = j * 2
        unpack_chunk(i0, ia)
        cpa = pltpu.async_copy(feats.at[ia.at[0]], rows_a, sem)
        unpack_chunk(i0 + 1, ib)
        cpb = pltpu.async_copy(feats.at[ib.at[0]], rows_b, sem)
        cpa.wait()
        pltpu.sync_copy(rows_a, acc_sh.at[ia.at[1]], add=True)
        cpb.wait()
        pltpu.sync_copy(rows_b, acc_sh.at[ib.at[1]], add=True)
        return ()

    lax.fori_loop(0, NCHUNK // 2, step, (), unroll=False)

    plsc.subcore_barrier()
    pltpu.sync_copy(acc_sh.at[pl.ds(r0, ROWS_PER_TILE)],
                    out.at[c, pl.ds(r0, ROWS_PER_TILE)])


@functools.cache
def _agg_sc():
    return pl.kernel(
        _agg_body,
        out_type=jax.ShapeDtypeStruct((NC, N_PAD, D), jnp.float32),
        mesh=plsc.VectorSubcoreMesh(core_axis_name="c", subcore_axis_name="s",
                                    num_cores=NC, num_subcores=NS),
        scratch_types=[
            pltpu.VMEM((NCHUNK, CH), jnp.int32),
            pltpu.VMEM((2, CH), jnp.int32),
            pltpu.VMEM((2, CH), jnp.int32),
            pltpu.VMEM((CH, D), jnp.float32),
            pltpu.VMEM((CH, D), jnp.float32),
            pltpu.VMEM_SHARED((N_PAD, D), jnp.float32),
            pltpu.SemaphoreType.DMA,
        ],
    )


BI = 640
GRID = (N + BI - 1) // BI


def _mlp_body(x_ref, agg_ref, eps_ref, w1_ref, b1_ref, w2_ref, b2_ref, h_ref):
    agg = agg_ref[0, :N, :] + agg_ref[1, :N, :]
    h0 = (1.0 + eps_ref[0, 0]) * x_ref[...] + agg
    h1 = jnp.maximum(
        jnp.dot(h0, w1_ref[...], preferred_element_type=jnp.float32)
        + b1_ref[...], 0.0)
    h2 = jnp.maximum(
        jnp.dot(h1, w2_ref[...], preferred_element_type=jnp.float32)
        + b2_ref[...], 0.0)
    h_ref[...] = h2.astype(jnp.bfloat16)


def _mlp_call(x, agg2, eps11, W1, b1r, W2, b2r):
    return pl.pallas_call(
        _mlp_body,
        out_shape=jax.ShapeDtypeStruct((N, D), jnp.bfloat16),
    )(x, agg2, eps11, W1, b1r, W2, b2r)


def _mm_body(h_full_ref, hb_ref, out_ref):
    out_ref[...] = lax.dot_general(hb_ref[...], h_full_ref[...],
                                   (((1,), (1,)), ((), ())),
                                   preferred_element_type=jnp.float32)


def _mm_call(h):
    return pl.pallas_call(
        _mm_body,
        grid=(GRID,),
        in_specs=[
            pl.BlockSpec((N, D), lambda i: (0, 0)),
            pl.BlockSpec((BI, D), lambda i: (i, 0)),
        ],
        out_specs=pl.BlockSpec((BI, N), lambda i: (i, 0)),
        out_shape=jax.ShapeDtypeStruct((N, N), jnp.float32),
    )(h, h)


def kernel(node_feats, edge_idx, eps, W1, b1, W2, b2):
    pad = E_PAD - E
    padv = jnp.arange(pad, dtype=jnp.int32)
    srcp = jnp.concatenate([edge_idx[0], padv % N])
    dstp = jnp.concatenate([edge_idx[1], N + padv % (N_PAD - N)])
    idx3 = (srcp | (dstp << 16)).reshape(NW, NCHUNK, CH)
    agg2 = _agg_sc()(node_feats, idx3)
    h = _mlp_call(node_feats, agg2, eps.reshape(1, 1), W1,
                  b1.reshape(1, D), W2, b2.reshape(1, D))
    return _mm_call(h)


# speedup vs baseline: 1.1800x; 1.1800x over previous
import functools

import jax
import jax.numpy as jnp
from jax import lax
from jax.experimental import pallas as pl
from jax.experimental.pallas import tpu as pltpu
from jax.experimental.pallas import tpu_sc as plsc

N, E, D = 10000, 320000, 128

NC, NS = 2, 16
NW = NC * NS
CH = 64
NCHUNK = 160
E_PAD = NW * NCHUNK * CH
N_PAD = 10240
ROWS_PER_TILE = N_PAD // NS


def _agg_body(feats, idx3, out, idxp, ir0, ir1, ir2, ir3, ra, rb, rc, rd,
              acc_sh, sem_g, sem_s):
    rows = [ra, rb, rc, rd]
    irs = [ir0, ir1, ir2, ir3]
    cid = lax.axis_index("c")
    sid = lax.axis_index("s")
    wid = cid * NS + sid
    r0 = pl.multiple_of(sid * ROWS_PER_TILE, 8)

    zv = jnp.zeros((16,), jnp.float32)
    for k in range(16):
        for m in range(D // 16):
            ra[k, pl.ds(m * 16, 16)] = zv

    def zstep(k, _):
        pltpu.sync_copy(ra.at[pl.ds(0, 16)],
                        acc_sh.at[pl.ds(r0 + k * 16, 16)])
        return ()

    lax.fori_loop(0, ROWS_PER_TILE // 16, zstep, (), unroll=False)

    pltpu.sync_copy(idx3.at[wid], idxp)
    plsc.subcore_barrier()

    def unpack(row, colpar, buf):
        for m in range(CH // 16):
            v = idxp[row, pl.ds(colpar * CH + m * 16, 16)]
            buf[0, pl.ds(m * 16, 16)] = lax.bitwise_and(v, jnp.int32(0xFFFF))
            buf[1, pl.ds(m * 16, 16)] = lax.shift_right_logical(
                v, jnp.int32(16))

    def gather(buf, rbuf):
        pltpu.async_copy(feats.at[buf.at[0]], rbuf, sem_g)

    def scatter(rbuf, buf):
        pltpu.async_copy(rbuf, acc_sh.at[buf.at[1]], sem_s, add=True)

    def drain(sem, rbuf):
        pltpu.make_async_copy(feats.at[pl.ds(0, CH)], rbuf, sem).wait()

    for p in range(3):
        unpack(p // 2, p % 2, irs[p])
        gather(irs[p], rows[p])

    def step(j, _):
        for p in range(4):
            cc = j * 4 + p
            drain(sem_g, rows[p])
            scatter(rows[p], irs[p])

            @pl.when(cc > 0)
            def _():
                drain(sem_s, rows[(p + 3) % 4])

            nxt = cc + 3

            @pl.when(nxt < NCHUNK)
            def _():
                unpack(j * 2 + (p + 3) // 2, (p + 3) % 2, irs[(p + 3) % 4])
                gather(irs[(p + 3) % 4], rows[(p + 3) % 4])
            del _
        return ()

    lax.fori_loop(0, NCHUNK // 4, step, (), unroll=False)

    drain(sem_s, rows[3])
    plsc.subcore_barrier()
    pltpu.sync_copy(acc_sh.at[pl.ds(r0, ROWS_PER_TILE)],
                    out.at[cid, pl.ds(r0, ROWS_PER_TILE)])


@functools.cache
def _agg_sc():
    return pl.kernel(
        _agg_body,
        out_type=jax.ShapeDtypeStruct((NC, N_PAD, D), jnp.float32),
        mesh=plsc.VectorSubcoreMesh(core_axis_name="c", subcore_axis_name="s",
                                    num_cores=NC, num_subcores=NS),
        scratch_types=[
            pltpu.VMEM((NCHUNK // 2, 2 * CH), jnp.int32),
            pltpu.VMEM((2, CH), jnp.int32),
            pltpu.VMEM((2, CH), jnp.int32),
            pltpu.VMEM((2, CH), jnp.int32),
            pltpu.VMEM((2, CH), jnp.int32),
            pltpu.VMEM((CH, D), jnp.float32),
            pltpu.VMEM((CH, D), jnp.float32),
            pltpu.VMEM((CH, D), jnp.float32),
            pltpu.VMEM((CH, D), jnp.float32),
            pltpu.VMEM_SHARED((N_PAD, D), jnp.float32),
            pltpu.SemaphoreType.DMA,
            pltpu.SemaphoreType.DMA,
        ],
    )


BI = 400
GRID = N // BI


def _tc_body(x_ref, agg_ref, eps_ref, w1_ref, b1_ref, w2_ref, b2_ref,
             out_ref, h_ref):
    i = pl.program_id(0)

    @pl.when(i == 0)
    def _():
        agg = agg_ref[0, :N, :] + agg_ref[1, :N, :]
        h0 = (1.0 + eps_ref[0, 0]) * x_ref[...] + agg
        h1 = jnp.maximum(
            jnp.dot(h0, w1_ref[...], preferred_element_type=jnp.float32)
            + b1_ref[...], 0.0)
        h2 = jnp.maximum(
            jnp.dot(h1, w2_ref[...], preferred_element_type=jnp.float32)
            + b2_ref[...], 0.0)
        h_ref[...] = h2.astype(jnp.bfloat16)

    hb = h_ref[pl.ds(i * BI, BI), :]
    out_ref[...] = lax.dot_general(hb, h_ref[...], (((1,), (1,)), ((), ())),
                                   preferred_element_type=jnp.float32)


def _tc_call(x, agg2, eps11, W1, b1r, W2, b2r):
    full = lambda shape: pl.BlockSpec(shape, lambda i: (0,) * len(shape))
    return pl.pallas_call(
        _tc_body,
        grid=(GRID,),
        in_specs=[
            full((N, D)),
            full((NC, N_PAD, D)),
            full((1, 1)),
            full((D, D)),
            full((1, D)),
            full((D, D)),
            full((1, D)),
        ],
        out_specs=pl.BlockSpec((BI, N), lambda i: (i, 0)),
        out_shape=jax.ShapeDtypeStruct((N, N), jnp.float32),
        scratch_shapes=[pltpu.VMEM((N, D), jnp.bfloat16)],
    )(x, agg2, eps11, W1, b1r, W2, b2r)


def kernel(node_feats, edge_idx, eps, W1, b1, W2, b2):
    pad = E_PAD - E
    padv = jnp.arange(pad, dtype=jnp.int32)
    srcp = jnp.concatenate([edge_idx[0], padv % N])
    dstp = jnp.concatenate([edge_idx[1], N + padv % (N_PAD - N)])
    idx3 = (srcp | (dstp << 16)).reshape(NW, NCHUNK // 2, 2 * CH)
    agg2 = _agg_sc()(node_feats, idx3)
    return _tc_call(node_feats, agg2, eps.reshape(1, 1), W1,
                    b1.reshape(1, D), W2, b2.reshape(1, D))


# speedup vs baseline: 1.1808x; 1.0007x over previous
import functools

import jax
import jax.numpy as jnp
from jax import lax
from jax.experimental import pallas as pl
from jax.experimental.pallas import tpu as pltpu
from jax.experimental.pallas import tpu_sc as plsc

N, E, D = 10000, 320000, 128

NC, NS = 2, 16
NW = NC * NS
CH = 64
NCHUNK = 160
E_PAD = NW * NCHUNK * CH
N_PAD = 10240
ROWS_PER_TILE = N_PAD // NS


def _agg_body(feats, idx3, out, idxp, ir0, ir1, ir2, ir3, ra, rb, rc, rd,
              acc_sh, sem_g, sem_s):
    rows = [ra, rb, rc, rd]
    irs = [ir0, ir1, ir2, ir3]
    cid = lax.axis_index("c")
    sid = lax.axis_index("s")
    wid = cid * NS + sid
    r0 = pl.multiple_of(sid * ROWS_PER_TILE, 8)

    zv = jnp.zeros((16,), jnp.float32)
    for k in range(CH):
        for m in range(D // 16):
            ra[k, pl.ds(m * 16, 16)] = zv
    for k in range(ROWS_PER_TILE // CH):
        pltpu.async_copy(ra, acc_sh.at[pl.ds(r0 + k * CH, CH)], sem_s)

    pltpu.sync_copy(idx3.at[wid], idxp)
    for k in range(ROWS_PER_TILE // CH):
        pltpu.make_async_copy(feats.at[pl.ds(0, CH)], ra, sem_s).wait()
    plsc.subcore_barrier()

    def unpack(row, colpar, buf):
        for m in range(CH // 16):
            v = idxp[row, pl.ds(colpar * CH + m * 16, 16)]
            buf[0, pl.ds(m * 16, 16)] = lax.bitwise_and(v, jnp.int32(0xFFFF))
            buf[1, pl.ds(m * 16, 16)] = lax.shift_right_logical(
                v, jnp.int32(16))

    def gather(buf, rbuf):
        pltpu.async_copy(feats.at[buf.at[0]], rbuf, sem_g)

    def scatter(rbuf, buf):
        pltpu.async_copy(rbuf, acc_sh.at[buf.at[1]], sem_s, add=True)

    def drain(sem, rbuf):
        pltpu.make_async_copy(feats.at[pl.ds(0, CH)], rbuf, sem).wait()

    for p in range(3):
        unpack(p // 2, p % 2, irs[p])
        gather(irs[p], rows[p])

    def step(j, _):
        for p in range(4):
            cc = j * 4 + p
            drain(sem_g, rows[p])
            scatter(rows[p], irs[p])

            @pl.when(cc > 0)
            def _():
                drain(sem_s, rows[(p + 3) % 4])

            nxt = cc + 3

            @pl.when(nxt < NCHUNK)
            def _():
                unpack(j * 2 + (p + 3) // 2, (p + 3) % 2, irs[(p + 3) % 4])
                gather(irs[(p + 3) % 4], rows[(p + 3) % 4])
            del _
        return ()

    lax.fori_loop(0, NCHUNK // 4, step, (), unroll=False)

    drain(sem_s, rows[3])
    plsc.subcore_barrier()
    pltpu.sync_copy(acc_sh.at[pl.ds(r0, ROWS_PER_TILE)],
                    out.at[cid, pl.ds(r0, ROWS_PER_TILE)])


@functools.cache
def _agg_sc():
    return pl.kernel(
        _agg_body,
        out_type=jax.ShapeDtypeStruct((NC, N_PAD, D), jnp.float32),
        mesh=plsc.VectorSubcoreMesh(core_axis_name="c", subcore_axis_name="s",
                                    num_cores=NC, num_subcores=NS),
        scratch_types=[
            pltpu.VMEM((NCHUNK // 2, 2 * CH), jnp.int32),
            pltpu.VMEM((2, CH), jnp.int32),
            pltpu.VMEM((2, CH), jnp.int32),
            pltpu.VMEM((2, CH), jnp.int32),
            pltpu.VMEM((2, CH), jnp.int32),
            pltpu.VMEM((CH, D), jnp.float32),
            pltpu.VMEM((CH, D), jnp.float32),
            pltpu.VMEM((CH, D), jnp.float32),
            pltpu.VMEM((CH, D), jnp.float32),
            pltpu.VMEM_SHARED((N_PAD, D), jnp.float32),
            pltpu.SemaphoreType.DMA,
            pltpu.SemaphoreType.DMA,
        ],
    )


BI = 400
GRID = N // BI


def _tc_body(x_ref, agg_ref, eps_ref, w1_ref, b1_ref, w2_ref, b2_ref,
             out_ref, h_ref):
    i = pl.program_id(0)

    @pl.when(i == 0)
    def _():
        agg = agg_ref[0, :N, :] + agg_ref[1, :N, :]
        h0 = (1.0 + eps_ref[0, 0]) * x_ref[...] + agg
        h1 = jnp.maximum(
            jnp.dot(h0, w1_ref[...], preferred_element_type=jnp.float32)
            + b1_ref[...], 0.0)
        h2 = jnp.maximum(
            jnp.dot(h1, w2_ref[...], preferred_element_type=jnp.float32)
            + b2_ref[...], 0.0)
        h_ref[...] = h2.astype(jnp.bfloat16)

    hb = h_ref[pl.ds(i * BI, BI), :]
    out_ref[...] = lax.dot_general(hb, h_ref[...], (((1,), (1,)), ((), ())),
                                   preferred_element_type=jnp.float32)


def _tc_call(x, agg2, eps11, W1, b1r, W2, b2r):
    full = lambda shape: pl.BlockSpec(shape, lambda i: (0,) * len(shape))
    return pl.pallas_call(
        _tc_body,
        grid=(GRID,),
        in_specs=[
            full((N, D)),
            full((NC, N_PAD, D)),
            full((1, 1)),
            full((D, D)),
            full((1, D)),
            full((D, D)),
            full((1, D)),
        ],
        out_specs=pl.BlockSpec((BI, N), lambda i: (i, 0)),
        out_shape=jax.ShapeDtypeStruct((N, N), jnp.float32),
        scratch_shapes=[pltpu.VMEM((N, D), jnp.bfloat16)],
    )(x, agg2, eps11, W1, b1r, W2, b2r)


def kernel(node_feats, edge_idx, eps, W1, b1, W2, b2):
    pad = E_PAD - E
    padv = jnp.arange(pad, dtype=jnp.int32)
    srcp = jnp.concatenate([edge_idx[0], padv % N])
    dstp = jnp.concatenate([edge_idx[1], N + padv % (N_PAD - N)])
    idx3 = (srcp | (dstp << 16)).reshape(NW, NCHUNK // 2, 2 * CH)
    agg2 = _agg_sc()(node_feats, idx3)
    return _tc_call(node_feats, agg2, eps.reshape(1, 1), W1,
                    b1.reshape(1, D), W2, b2.reshape(1, D))


# speedup vs baseline: 1.1884x; 1.0064x over previous
import functools

import jax
import jax.numpy as jnp
from jax import lax
from jax.experimental import pallas as pl
from jax.experimental.pallas import tpu as pltpu
from jax.experimental.pallas import tpu_sc as plsc

N, E, D = 10000, 320000, 128

NC, NS = 2, 16
NW = NC * NS
CH = 64
NCHUNK = 160
E_PAD = NW * NCHUNK * CH
N_PAD = 10240
ROWS_PER_TILE = N_PAD // NS


def _agg_body(feats, idx3, out, idxp, ir0, ir1, ir2, ir3, ra, rb, rc, rd,
              acc_sh, sem_g, sem_s):
    rows = [ra, rb, rc, rd]
    irs = [ir0, ir1, ir2, ir3]
    cid = lax.axis_index("c")
    sid = lax.axis_index("s")
    wid = cid * NS + sid
    r0 = pl.multiple_of(sid * ROWS_PER_TILE, 8)

    zv = jnp.zeros((16,), jnp.float32)
    for k in range(CH):
        for m in range(D // 16):
            ra[k, pl.ds(m * 16, 16)] = zv
    for k in range(ROWS_PER_TILE // CH):
        pltpu.async_copy(ra, acc_sh.at[pl.ds(r0 + k * CH, CH)], sem_s)

    pltpu.sync_copy(idx3.at[wid], idxp)
    for k in range(ROWS_PER_TILE // CH):
        pltpu.make_async_copy(feats.at[pl.ds(0, CH)], ra, sem_s).wait()
    plsc.subcore_barrier()

    def unpack(row, colpar, buf):
        for m in range(CH // 16):
            v = idxp[row, pl.ds(colpar * CH + m * 16, 16)]
            buf[0, pl.ds(m * 16, 16)] = lax.bitwise_and(v, jnp.int32(0xFFFF))
            buf[1, pl.ds(m * 16, 16)] = lax.shift_right_logical(
                v, jnp.int32(16))

    def gather(buf, rbuf):
        pltpu.async_copy(feats.at[buf.at[0]], rbuf, sem_g)

    def scatter(rbuf, buf):
        pltpu.async_copy(rbuf, acc_sh.at[buf.at[1]], sem_s, add=True)

    def drain(sem, rbuf):
        pltpu.make_async_copy(feats.at[pl.ds(0, CH)], rbuf, sem).wait()

    for p in range(3):
        unpack(p // 2, p % 2, irs[p])
        gather(irs[p], rows[p])

    def step(j, _):
        for p in range(4):
            cc = j * 4 + p
            drain(sem_g, rows[p])
            scatter(rows[p], irs[p])

            @pl.when(cc > 0)
            def _():
                drain(sem_s, rows[(p + 3) % 4])

            nxt = cc + 3

            @pl.when(nxt < NCHUNK)
            def _():
                unpack(j * 2 + (p + 3) // 2, (p + 3) % 2, irs[(p + 3) % 4])
                gather(irs[(p + 3) % 4], rows[(p + 3) % 4])
            del _
        return ()

    lax.fori_loop(0, NCHUNK // 4, step, (), unroll=False)

    drain(sem_s, rows[3])
    plsc.subcore_barrier()
    pltpu.sync_copy(acc_sh.at[pl.ds(r0, ROWS_PER_TILE)],
                    out.at[cid, pl.ds(r0, ROWS_PER_TILE)])


@functools.cache
def _agg_sc():
    return pl.kernel(
        _agg_body,
        out_type=jax.ShapeDtypeStruct((NC, N_PAD, D), jnp.float32),
        mesh=plsc.VectorSubcoreMesh(core_axis_name="c", subcore_axis_name="s",
                                    num_cores=NC, num_subcores=NS),
        scratch_types=[
            pltpu.VMEM((NCHUNK // 2, 2 * CH), jnp.int32),
            pltpu.VMEM((2, CH), jnp.int32),
            pltpu.VMEM((2, CH), jnp.int32),
            pltpu.VMEM((2, CH), jnp.int32),
            pltpu.VMEM((2, CH), jnp.int32),
            pltpu.VMEM((CH, D), jnp.float32),
            pltpu.VMEM((CH, D), jnp.float32),
            pltpu.VMEM((CH, D), jnp.float32),
            pltpu.VMEM((CH, D), jnp.float32),
            pltpu.VMEM_SHARED((N_PAD, D), jnp.float32),
            pltpu.SemaphoreType.DMA,
            pltpu.SemaphoreType.DMA,
        ],
    )


BI = 480
GRID = (N + BI - 1) // BI


def _tc_body(x_ref, agg_ref, eps_ref, w1_ref, b1_ref, w2_ref, b2_ref,
             out_ref, h_ref):
    i = pl.program_id(0)

    @pl.when(i == 0)
    def _():
        agg = agg_ref[0, :N, :] + agg_ref[1, :N, :]
        h0 = (1.0 + eps_ref[0, 0]) * x_ref[...] + agg
        h1 = jnp.maximum(
            jnp.dot(h0, w1_ref[...], preferred_element_type=jnp.float32)
            + b1_ref[...], 0.0)
        h2 = jnp.maximum(
            jnp.dot(h1, w2_ref[...], preferred_element_type=jnp.float32)
            + b2_ref[...], 0.0)
        h_ref[...] = h2.astype(jnp.bfloat16)

    hb = h_ref[pl.ds(i * BI, BI), :]
    out_ref[...] = lax.dot_general(hb, h_ref[...], (((1,), (1,)), ((), ())),
                                   preferred_element_type=jnp.float32)


def _tc_call(x, agg2, eps11, W1, b1r, W2, b2r):
    full = lambda shape: pl.BlockSpec(shape, lambda i: (0,) * len(shape))
    return pl.pallas_call(
        _tc_body,
        grid=(GRID,),
        in_specs=[
            full((N, D)),
            full((NC, N_PAD, D)),
            full((1, 1)),
            full((D, D)),
            full((1, D)),
            full((D, D)),
            full((1, D)),
        ],
        out_specs=pl.BlockSpec((BI, N), lambda i: (i, 0)),
        out_shape=jax.ShapeDtypeStruct((N, N), jnp.float32),
        scratch_shapes=[pltpu.VMEM((N, D), jnp.bfloat16)],
    )(x, agg2, eps11, W1, b1r, W2, b2r)


def kernel(node_feats, edge_idx, eps, W1, b1, W2, b2):
    pad = E_PAD - E
    padv = jnp.arange(pad, dtype=jnp.int32)
    srcp = jnp.concatenate([edge_idx[0], padv % N])
    dstp = jnp.concatenate([edge_idx[1], N + padv % (N_PAD - N)])
    idx3 = (srcp | (dstp << 16)).reshape(NW, NCHUNK // 2, 2 * CH)
    agg2 = _agg_sc()(node_feats, idx3)
    return _tc_call(node_feats, agg2, eps.reshape(1, 1), W1,
                    b1.reshape(1, D), W2, b2.reshape(1, D))
